# baseline (speedup 1.0000x reference)
import functools
import jax
import jax.numpy as jnp
from jax import lax
from jax.experimental import pallas as pl
from jax.experimental.pallas import tpu as pltpu
from jax.experimental.pallas import tpu_sc as plsc

f32 = jnp.float32
i32 = jnp.int32

N = 10000
E = 320000
NG = 64
H = 64
GW = 128
NW = 32
CH = 128
EPAD = NW * 80 * CH
NCH = 80
NACC = 10240
NPS = NACC // 16
NPOOL = 12288
PCH = 3


@functools.cache
def _sc_mesh():
    return plsc.VectorSubcoreMesh(core_axis_name="c", subcore_axis_name="s",
                                  num_cores=2, num_subcores=16)


def _silu(x):
    return x * jax.nn.sigmoid(x)


def _gather_body(a_hbm, b_hbm, row_hbm, col_hbm, out_hbm,
                 rowi, coli, bufa0, bufb0, bufa1, bufb1,
                 sem0, sem1, osem0, osem1):
    cid = lax.axis_index("c")
    sid = lax.axis_index("s")
    w = sid * 2 + cid
    base0 = w * (NCH * CH)

    pltpu.sync_copy(row_hbm.at[w], rowi)
    pltpu.sync_copy(col_hbm.at[w], coli)

    sets = ((bufa0, bufb0, sem0, osem0),
            (bufa1, bufb1, sem1, osem1))

    def stage(i, s):
        bufa, bufb, sem, osem = s
        pltpu.async_copy(a_hbm.at[rowi.at[i]], bufa, sem)

    def process(i, s):
        bufa, bufb, sem, osem = s
        base = base0 + i * CH
        pltpu.make_async_copy(a_hbm.at[rowi.at[i]], bufa, sem).wait()
        pltpu.async_copy(b_hbm.at[coli.at[i]], bufa, sem, add=True)
        pltpu.make_async_copy(b_hbm.at[coli.at[i]], bufa, sem).wait()
        pltpu.async_copy(bufa, out_hbm.at[pl.ds(base, CH)], osem)

    def waitout(i, s):
        bufa, bufb, sem, osem = s
        base = base0 + i * CH
        pltpu.make_async_copy(bufa, out_hbm.at[pl.ds(base, CH)], osem).wait()

    stage(0, sets[0])

    @pl.loop(0, NCH, step=2)
    def _(i):
        @pl.when(i + 1 < NCH)
        def _():
            stage(i + 1, sets[1])
        process(i, sets[0])

        @pl.when(i + 2 < NCH)
        def _():
            waitout(i, sets[0])
            stage(i + 2, sets[0])

        @pl.when(i + 1 < NCH)
        def _():
            process(i + 1, sets[1])

        @pl.when(i + 3 < NCH)
        def _():
            waitout(i + 1, sets[1])

    waitout(NCH - 2, sets[0])
    waitout(NCH - 1, sets[1])


def _gather_edges(a, b, row3, col3):
    return pl.kernel(
        _gather_body,
        out_type=jax.ShapeDtypeStruct((EPAD, GW), f32),
        mesh=_sc_mesh(),
        scratch_types=[
            pltpu.VMEM((NCH, CH), i32), pltpu.VMEM((NCH, CH), i32),
            pltpu.VMEM((CH, GW), f32), pltpu.VMEM((CH, GW), f32),
            pltpu.VMEM((CH, GW), f32), pltpu.VMEM((CH, GW), f32),
            pltpu.SemaphoreType.DMA, pltpu.SemaphoreType.DMA,
            pltpu.SemaphoreType.DMA, pltpu.SemaphoreType.DMA,
        ],
        name="sc_gather_edges",
    )(a, b, row3, col3)


def _scatter_body(ety_hbm, row_hbm, z_hbm, part_hbm,
                  rowi, vbuf0, vbuf1,
                  acc, sem0, sem1, ssem0, ssem1):
    cid = lax.axis_index("c")
    sid = lax.axis_index("s")
    w = sid * 2 + cid
    base0 = w * (NCH * CH)

    pltpu.sync_copy(row_hbm.at[w], rowi)
    pltpu.sync_copy(z_hbm, acc.at[pl.ds(sid * NPS, NPS)])
    plsc.subcore_barrier()

    sets = ((vbuf0, sem0, ssem0), (vbuf1, sem1, ssem1))

    def stage(i, s):
        vbuf, sem, ssem = s
        base = base0 + i * CH
        pltpu.async_copy(ety_hbm.at[pl.ds(base, CH)], vbuf, sem)

    def process(i, s):
        vbuf, sem, ssem = s
        base = base0 + i * CH
        pltpu.make_async_copy(ety_hbm.at[pl.ds(base, CH)], vbuf, sem).wait()
        pltpu.async_copy(vbuf, acc.at[rowi.at[i]], ssem, add=True)

    def waitsc(i, s):
        vbuf, sem, ssem = s
        pltpu.make_async_copy(vbuf, acc.at[rowi.at[i]], ssem).wait()

    stage(0, sets[0])

    @pl.loop(0, NCH, step=2)
    def _(i):
        @pl.when(i + 1 < NCH)
        def _():
            stage(i + 1, sets[1])
        process(i, sets[0])

        @pl.when(i + 2 < NCH)
        def _():
            waitsc(i, sets[0])
            stage(i + 2, sets[0])

        @pl.when(i + 1 < NCH)
        def _():
            process(i + 1, sets[1])

        @pl.when(i + 3 < NCH)
        def _():
            waitsc(i + 1, sets[1])

    waitsc(NCH - 2, sets[0])
    waitsc(NCH - 1, sets[1])
    plsc.subcore_barrier()

    pltpu.sync_copy(acc.at[pl.ds(sid * NPS, NPS)],
                    part_hbm.at[cid, pl.ds(sid * NPS, NPS)])


def _scatter_edges(ety, row3, z):
    return pl.kernel(
        _scatter_body,
        out_type=jax.ShapeDtypeStruct((2, NACC, GW), f32),
        mesh=_sc_mesh(),
        scratch_types=[
            pltpu.VMEM((NCH, CH), i32),
            pltpu.VMEM((CH, GW), f32), pltpu.VMEM((CH, GW), f32),
            pltpu.VMEM_SHARED((NACC, GW), f32),
            pltpu.SemaphoreType.DMA, pltpu.SemaphoreType.DMA,
            pltpu.SemaphoreType.DMA, pltpu.SemaphoreType.DMA,
        ],
        name="sc_scatter_edges",
    )(ety, row3, z)


def _pool_body(vals_hbm, idx_hbm, z_hbm, out_hbm, idxv, vbuf, acc, sem):
    cid = lax.axis_index("c")
    sid = lax.axis_index("s")
    w = sid * 2 + cid
    base0 = w * (PCH * CH)

    @pl.when(sid == 0)
    def _():
        pltpu.sync_copy(z_hbm, acc)
    plsc.subcore_barrier()

    pltpu.sync_copy(idx_hbm.at[w], idxv)

    def step(i, _):
        base = base0 + i * CH
        pltpu.sync_copy(vals_hbm.at[pl.ds(base, CH)], vbuf)
        pltpu.sync_copy(vbuf, acc.at[idxv.at[i]], add=True)
        return 0
    lax.fori_loop(0, PCH, step, 0)
    plsc.subcore_barrier()

    @pl.when(sid == 0)
    def _():
        pltpu.sync_copy(acc, out_hbm.at[cid])


def _pool_scatter(vals, idx, z):
    return pl.kernel(
        _pool_body,
        out_type=jax.ShapeDtypeStruct((2, NG, GW), f32),
        mesh=_sc_mesh(),
        scratch_types=[
            pltpu.VMEM((PCH, CH), i32), pltpu.VMEM((CH, GW), f32),
            pltpu.VMEM_SHARED((NG, GW), f32),
            pltpu.SemaphoreType.DMA,
        ],
        name="sc_pool_scatter",
    )(vals, idx, z)


NB = 1000
EB = 2048


def _full(shape):
    return pl.BlockSpec(shape, lambda i: (0,) * len(shape))


def _emb_body(x_ref, w_ref, b_ref, o_ref, *, mode):
    y = jnp.dot(x_ref[...], w_ref[...], preferred_element_type=f32) + b_ref[...]
    nb, dout = y.shape
    if mode == 'pad128':
        y = jnp.concatenate([y, jnp.zeros((nb, GW - dout), f32)], axis=1)
    elif mode == 'ones128':
        y = jnp.concatenate([y, jnp.ones((nb, 8), f32),
                             jnp.zeros((nb, GW - dout - 8), f32)], axis=1)
    o_ref[...] = y


def _emb(x, w, b, mode=None):
    din, dout = w.shape
    dw = dout if mode is None else GW
    return pl.pallas_call(
        functools.partial(_emb_body, mode=mode),
        grid=(N // NB,),
        in_specs=[pl.BlockSpec((NB, din), lambda i: (i, 0)),
                  _full((din, dout)), _full((1, dout))],
        out_specs=pl.BlockSpec((NB, dw), lambda i: (i, 0)),
        out_shape=jax.ShapeDtypeStruct((N, dw), f32),
        name="tc_emb",
    )(x, w, b.reshape(1, -1))


def _prep_body(h_ref, xp_ref, wa_ref, wb_ref, b1_ref, a_ref, b_ref):
    h = h_ref[...]
    xp = xp_ref[...]
    z = jnp.zeros((h.shape[0], GW - H - 8), f32)
    ha = jnp.dot(h, wa_ref[...], preferred_element_type=f32) + b1_ref[...]
    hb = jnp.dot(h, wb_ref[...], preferred_element_type=f32)
    a_ref[...] = jnp.concatenate([ha, xp, z], axis=1)
    b_ref[...] = jnp.concatenate([hb, -xp, z], axis=1)


def _prep(h, xp, wa, wb, b1):
    return pl.pallas_call(
        _prep_body,
        grid=(N // NB,),
        in_specs=[pl.BlockSpec((NB, H), lambda i: (i, 0)),
                  pl.BlockSpec((NB, 8), lambda i: (i, 0)),
                  _full((H, H)), _full((H, H)), _full((1, H))],
        out_specs=[pl.BlockSpec((NB, GW), lambda i: (i, 0)),
                   pl.BlockSpec((NB, GW), lambda i: (i, 0))],
        out_shape=[jax.ShapeDtypeStruct((N, GW), f32),
                   jax.ShapeDtypeStruct((N, GW), f32)],
        name="tc_prep",
    )(h, xp, wa, wb, b1.reshape(1, -1))


def _edge_body(ex_ref, w1c_ref, w2_ref, b2_ref, watt_ref, batt_ref,
               wc1_ref, bc1_ref, wc2_ref, ety_ref):
    ex = ex_ref[...]
    epre = ex[:, :H]
    cd8 = ex[:, H:H + 8]
    radial = jnp.sum(cd8 * cd8, axis=1, keepdims=True)
    e1 = _silu(epre + radial * w1c_ref[...])
    e2 = _silu(jnp.dot(e1, w2_ref[...], preferred_element_type=f32)
               + b2_ref[...])
    att = jax.nn.sigmoid(jnp.sum(e2 * watt_ref[...], axis=1, keepdims=True)
                         + batt_ref[...])
    eo = e2 * att
    c1 = _silu(jnp.dot(eo, wc1_ref[...], preferred_element_type=f32)
               + bc1_ref[...])
    t = jnp.tanh(jnp.sum(c1 * wc2_ref[...], axis=1, keepdims=True))
    col8 = lax.broadcasted_iota(i32, (1, 8), 1)
    tr8 = cd8 * t + jnp.where(col8 == 3, 1.0, 0.0)
    z = jnp.zeros((eo.shape[0], GW - H - 8), f32)
    ety_ref[...] = jnp.concatenate([eo, tr8, z], axis=1)


def _edge(ex, w1c128, w2p, b2, watt, batt, wc1, bc1, wc2):
    return pl.pallas_call(
        _edge_body,
        grid=(EPAD // EB,),
        in_specs=[pl.BlockSpec((EB, GW), lambda i: (i, 0)),
                  _full((1, H)), _full((H, H)), _full((1, H)),
                  _full((1, H)), _full((1, 1)),
                  _full((H, H)), _full((1, H)), _full((1, H))],
        out_specs=pl.BlockSpec((EB, GW), lambda i: (i, 0)),
        out_shape=jax.ShapeDtypeStruct((EPAD, GW), f32),
        name="tc_edge",
    )(ex, w1c128, w2p, b2, watt, batt, wc1, bc1, wc2)


def _node_body(h_ref, xp_ref, p_ref, w1h_ref, w1a_ref, b1_ref,
               w2_ref, b2_ref, ho_ref, xo_ref):
    h = h_ref[...]
    p = p_ref[0] + p_ref[1]
    tp = p[:, H:H + 8]
    col8 = lax.broadcasted_iota(i32, (1, 8), 1)
    cnt = jnp.sum(jnp.where(col8 == 3, tp, 0.0), axis=1, keepdims=True)
    cnt = jnp.maximum(cnt, 1.0)
    xo_ref[...] = xp_ref[...] + jnp.where(col8 < 3, tp, 0.0) / cnt
    m1 = _silu(jnp.dot(h, w1h_ref[...], preferred_element_type=f32)
               + jnp.dot(p, w1a_ref[...], preferred_element_type=f32)
               + b1_ref[...])
    ho_ref[...] = h + jnp.dot(m1, w2_ref[...], preferred_element_type=f32) \
        + b2_ref[...]


def _node(h, xp, part, w1h, w1a128, b1, w2, b2):
    return pl.pallas_call(
        _node_body,
        grid=(N // NB,),
        in_specs=[pl.BlockSpec((NB, H), lambda i: (i, 0)),
                  pl.BlockSpec((NB, 8), lambda i: (i, 0)),
                  pl.BlockSpec((2, NB, GW), lambda i: (0, i, 0)),
                  _full((H, H)), _full((GW, H)), _full((1, H)),
                  _full((H, H)), _full((1, H))],
        out_specs=[pl.BlockSpec((NB, H), lambda i: (i, 0)),
                   pl.BlockSpec((NB, 8), lambda i: (i, 0))],
        out_shape=[jax.ShapeDtypeStruct((N, H), f32),
                   jax.ShapeDtypeStruct((N, 8), f32)],
        name="tc_node",
    )(h, xp, part, w1h, w1a128, b1.reshape(1, -1), w2, b2.reshape(1, -1))


def _bn_mat(g, b, m):
    mu = jnp.mean(m, axis=0, keepdims=True)
    var = jnp.mean((m - mu) ** 2, axis=0, keepdims=True)
    return g * (m - mu) * jax.lax.rsqrt(var + 1e-5) + b


def _head_body(p1_ref, p2_ref, p3_ref,
               g1_ref, b1_ref, g2_ref, b2_ref, g3_ref, b3_ref,
               wp1_ref, wp2_ref, wp3_ref, bfc_ref, gf_ref, bf_ref,
               wf_ref, bff_ref, o_ref):
    p1 = p1_ref[0] + p1_ref[1]
    p2 = p2_ref[0] + p2_ref[1]
    p3 = p3_ref[0] + p3_ref[1]
    col = lax.broadcasted_iota(i32, (1, GW), 1)
    cnt = jnp.sum(jnp.where(col == 32, p3, 0.0), axis=1, keepdims=True)
    cnt = jnp.maximum(cnt, 1.0)
    m1 = jax.nn.relu(_bn_mat(g1_ref[...], b1_ref[...], p1 / cnt))
    m2 = jax.nn.relu(_bn_mat(g2_ref[...], b2_ref[...], p2 / cnt))
    m3 = jax.nn.relu(_bn_mat(g3_ref[...], b3_ref[...], p3 / cnt))
    u = (jnp.dot(m1, wp1_ref[...], preferred_element_type=f32)
         + jnp.dot(m2, wp2_ref[...], preferred_element_type=f32)
         + jnp.dot(m3, wp3_ref[...], preferred_element_type=f32)
         + bfc_ref[...])
    u = _bn_mat(gf_ref[...], bf_ref[...], u)
    o_ref[...] = jnp.dot(u, wf_ref[...], preferred_element_type=f32) \
        + bff_ref[...]


def _head(p1, p2, p3, bn1, bn2, bn3, wp1, wp2, wp3, bfc, bnfc, wf, bff):
    df, do = 178, 128
    return pl.pallas_call(
        _head_body,
        grid=(1,),
        in_specs=[_full((2, NG, GW)), _full((2, NG, GW)), _full((2, NG, GW)),
                  _full((1, GW)), _full((1, GW)),
                  _full((1, GW)), _full((1, GW)),
                  _full((1, GW)), _full((1, GW)),
                  _full((GW, df)), _full((GW, df)), _full((GW, df)),
                  _full((1, df)), _full((1, df)), _full((1, df)),
                  _full((df, do)), _full((1, do))],
        out_specs=_full((NG, do)),
        out_shape=jax.ShapeDtypeStruct((NG, do), f32),
        name="tc_head",
    )(p1, p2, p3, bn1[0], bn1[1], bn2[0], bn2[1], bn3[0], bn3[1],
      wp1, wp2, wp3, bfc, bnfc[0], bnfc[1], wf, bff)


def _pad128(v, rows=None):
    out = jnp.zeros((rows or v.shape[0], GW), f32)
    return out.at[:v.shape[0], :v.shape[1]].set(v)


def _egnn_block(p, h_in, xp, row_g, col_g, row_s, zacc):
    h = _emb(h_in, p['emb_in']['W'], p['emb_in']['b'])
    for lp in p['layers']:
        w1 = lp['e1']['W']
        a, b = _prep(h, xp, w1[:H], w1[H:2 * H], lp['e1']['b'])
        ex = _gather_edges(a, b, row_g, col_g)
        ety = _edge(ex, w1[2 * H:2 * H + 1], lp['e2']['W'],
                    lp['e2']['b'].reshape(1, -1),
                    lp['att']['W'].reshape(1, -1),
                    lp['att']['b'].reshape(1, 1),
                    lp['c1']['W'], lp['c1']['b'].reshape(1, -1),
                    lp['c2']['W'].reshape(1, -1))
        part = _scatter_edges(ety, row_s, zacc)
        wn = lp['n1']['W']
        w1a128 = jnp.concatenate([wn[H:], jnp.zeros((GW - H, H), f32)], axis=0)
        h, xp = _node(h, xp, part, wn[:H], w1a128, lp['n1']['b'],
                      lp['n2']['W'], lp['n2']['b'])
    return h, xp


def kernel(x, coords, params, edge_index, batch):
    row, col = edge_index[0], edge_index[1]
    epad = EPAD - E
    row_g = jnp.concatenate([row, jnp.zeros((epad,), i32)]
                            ).reshape(NW, NCH, CH)
    col_g = jnp.concatenate([col, jnp.zeros((epad,), i32)]
                            ).reshape(NW, NCH, CH)
    row_s = jnp.concatenate([row, jnp.full((epad,), N + 8, i32)]
                            ).reshape(NW, NCH, CH)
    xp = jnp.pad(coords, ((0, 0), (0, 5)))
    zacc = jnp.zeros((NPS, GW), f32)
    zpool = jnp.zeros((NG, GW), f32)
    bpad = jnp.concatenate([batch, jnp.zeros((NPOOL - N,), i32)]
                           ).reshape(NW, PCH, CH)

    p = params
    h1, xp1 = _egnn_block(p['egnn1'], x, xp, row_g, col_g, row_s, zacc)
    h1 = _emb(h1, p['egnn1']['emb_out']['W'], p['egnn1']['emb_out']['b'])
    h2, xp2 = _egnn_block(p['egnn2'], h1, xp1, row_g, col_g, row_s, zacc)
    h2p = _emb(h2, p['egnn2']['emb_out']['W'], p['egnn2']['emb_out']['b'],
               mode='pad128')
    h3, _ = _egnn_block(p['egnn4'], h2p[:, :H], xp2, row_g, col_g, row_s,
                        zacc)
    h3p = _emb(h3, p['egnn4']['emb_out']['W'], p['egnn4']['emb_out']['b'],
               mode='ones128')

    v1 = jnp.zeros((NPOOL, GW), f32).at[:N].set(h1)
    v2 = jnp.zeros((NPOOL, GW), f32).at[:N].set(h2p)
    v3 = jnp.zeros((NPOOL, GW), f32).at[:N].set(h3p)
    p1 = _pool_scatter(v1, bpad, zpool)
    p2 = _pool_scatter(v2, bpad, zpool)
    p3 = _pool_scatter(v3, bpad, zpool)

    w = p['fc1']['W']
    wp1 = w[:128]
    wp2 = jnp.concatenate([w[128:192], jnp.zeros((GW - 64, 178), f32)])
    wp3 = jnp.concatenate([w[192:224], jnp.zeros((GW - 32, 178), f32)])
    out = _head(p1, p2, p3,
                (_pad128(p['bn1']['g'].reshape(1, -1), 1),
                 _pad128(p['bn1']['b'].reshape(1, -1), 1)),
                (_pad128(p['bn2']['g'].reshape(1, -1), 1),
                 _pad128(p['bn2']['b'].reshape(1, -1), 1)),
                (_pad128(p['bn3']['g'].reshape(1, -1), 1),
                 _pad128(p['bn3']['b'].reshape(1, -1), 1)),
                wp1, wp2, wp3, p['fc1']['b'].reshape(1, -1),
                (p['bn_fc1']['g'].reshape(1, -1),
                 p['bn_fc1']['b'].reshape(1, -1)),
                p['final']['W'], p['final']['b'].reshape(1, -1))
    return out


# speedup vs baseline: 1.0145x; 1.0145x over previous
import functools
import jax
import jax.numpy as jnp
from jax import lax
from jax.experimental import pallas as pl
from jax.experimental.pallas import tpu as pltpu
from jax.experimental.pallas import tpu_sc as plsc

f32 = jnp.float32
i32 = jnp.int32

N = 10000
E = 320000
NG = 64
H = 64
GW = 128
NW = 32
CH = 128
EPAD = NW * 80 * CH
NCH = 80
NACC = 10240
NPS = NACC // 16
NPOOL = 12288
PCH = 3


@functools.cache
def _sc_mesh():
    return plsc.VectorSubcoreMesh(core_axis_name="c", subcore_axis_name="s",
                                  num_cores=2, num_subcores=16)


def _silu(x):
    return x * jax.nn.sigmoid(x)


def _gather_body(a_hbm, b_hbm, row_hbm, col_hbm, out_hbm,
                 rowi, coli, bufa0, bufb0, bufa1, bufb1,
                 sem0, sem1, osem0, osem1):
    cid = lax.axis_index("c")
    sid = lax.axis_index("s")
    w = sid * 2 + cid
    base0 = w * (NCH * CH)

    pltpu.sync_copy(row_hbm.at[w], rowi)
    pltpu.sync_copy(col_hbm.at[w], coli)

    sets = ((bufa0, bufb0, sem0, osem0),
            (bufa1, bufb1, sem1, osem1))

    def stage(i, s):
        bufa, bufb, sem, osem = s
        pltpu.async_copy(a_hbm.at[rowi.at[i]], bufa, sem)
        pltpu.async_copy(b_hbm.at[coli.at[i]], bufb, sem)

    def process(i, s):
        bufa, bufb, sem, osem = s
        base = base0 + i * CH
        pltpu.make_async_copy(a_hbm.at[rowi.at[i]], bufa, sem).wait()
        pltpu.make_async_copy(b_hbm.at[coli.at[i]], bufb, sem).wait()

        @pl.loop(0, CH, unroll=8)
        def _(r):
            for c in range(5):
                bufa[r, pl.ds(c * 16, 16)] = (bufa[r, pl.ds(c * 16, 16)] +
                                              bufb[r, pl.ds(c * 16, 16)])
        pltpu.async_copy(bufa, out_hbm.at[pl.ds(base, CH)], osem)

    def waitout(i, s):
        bufa, bufb, sem, osem = s
        base = base0 + i * CH
        pltpu.make_async_copy(bufa, out_hbm.at[pl.ds(base, CH)], osem).wait()

    stage(0, sets[0])

    @pl.loop(0, NCH, step=2)
    def _(i):
        @pl.when(i + 1 < NCH)
        def _():
            stage(i + 1, sets[1])
        process(i, sets[0])

        @pl.when(i + 2 < NCH)
        def _():
            waitout(i, sets[0])
            stage(i + 2, sets[0])

        @pl.when(i + 1 < NCH)
        def _():
            process(i + 1, sets[1])

        @pl.when(i + 3 < NCH)
        def _():
            waitout(i + 1, sets[1])

    waitout(NCH - 2, sets[0])
    waitout(NCH - 1, sets[1])


def _gather_edges(a, b, row3, col3):
    return pl.kernel(
        _gather_body,
        out_type=jax.ShapeDtypeStruct((EPAD, GW), f32),
        mesh=_sc_mesh(),
        scratch_types=[
            pltpu.VMEM((NCH, CH), i32), pltpu.VMEM((NCH, CH), i32),
            pltpu.VMEM((CH, GW), f32), pltpu.VMEM((CH, GW), f32),
            pltpu.VMEM((CH, GW), f32), pltpu.VMEM((CH, GW), f32),
            pltpu.SemaphoreType.DMA, pltpu.SemaphoreType.DMA,
            pltpu.SemaphoreType.DMA, pltpu.SemaphoreType.DMA,
        ],
        name="sc_gather_edges",
    )(a, b, row3, col3)


def _scatter_body(ety_hbm, row_hbm, z_hbm, part_hbm,
                  rowi, vbuf0, vbuf1,
                  acc, sem0, sem1, ssem0, ssem1):
    cid = lax.axis_index("c")
    sid = lax.axis_index("s")
    w = sid * 2 + cid
    base0 = w * (NCH * CH)

    pltpu.sync_copy(row_hbm.at[w], rowi)
    pltpu.sync_copy(z_hbm, acc.at[pl.ds(sid * NPS, NPS)])
    plsc.subcore_barrier()

    sets = ((vbuf0, sem0, ssem0), (vbuf1, sem1, ssem1))

    def stage(i, s):
        vbuf, sem, ssem = s
        base = base0 + i * CH
        pltpu.async_copy(ety_hbm.at[pl.ds(base, CH)], vbuf, sem)

    def process(i, s):
        vbuf, sem, ssem = s
        base = base0 + i * CH
        pltpu.make_async_copy(ety_hbm.at[pl.ds(base, CH)], vbuf, sem).wait()
        pltpu.async_copy(vbuf, acc.at[rowi.at[i]], ssem, add=True)

    def waitsc(i, s):
        vbuf, sem, ssem = s
        pltpu.make_async_copy(vbuf, acc.at[rowi.at[i]], ssem).wait()

    stage(0, sets[0])

    @pl.loop(0, NCH, step=2)
    def _(i):
        @pl.when(i + 1 < NCH)
        def _():
            stage(i + 1, sets[1])
        process(i, sets[0])

        @pl.when(i + 2 < NCH)
        def _():
            waitsc(i, sets[0])
            stage(i + 2, sets[0])

        @pl.when(i + 1 < NCH)
        def _():
            process(i + 1, sets[1])

        @pl.when(i + 3 < NCH)
        def _():
            waitsc(i + 1, sets[1])

    waitsc(NCH - 2, sets[0])
    waitsc(NCH - 1, sets[1])
    plsc.subcore_barrier()

    pltpu.sync_copy(acc.at[pl.ds(sid * NPS, NPS)],
                    part_hbm.at[cid, pl.ds(sid * NPS, NPS)])


def _scatter_edges(ety, row3, z):
    return pl.kernel(
        _scatter_body,
        out_type=jax.ShapeDtypeStruct((2, NACC, GW), f32),
        mesh=_sc_mesh(),
        scratch_types=[
            pltpu.VMEM((NCH, CH), i32),
            pltpu.VMEM((CH, GW), f32), pltpu.VMEM((CH, GW), f32),
            pltpu.VMEM_SHARED((NACC, GW), f32),
            pltpu.SemaphoreType.DMA, pltpu.SemaphoreType.DMA,
            pltpu.SemaphoreType.DMA, pltpu.SemaphoreType.DMA,
        ],
        name="sc_scatter_edges",
    )(ety, row3, z)


def _pool_body(vals_hbm, idx_hbm, z_hbm, out_hbm, idxv, vbuf, acc, sem):
    cid = lax.axis_index("c")
    sid = lax.axis_index("s")
    w = sid * 2 + cid
    base0 = w * (PCH * CH)

    @pl.when(sid == 0)
    def _():
        pltpu.sync_copy(z_hbm, acc)
    plsc.subcore_barrier()

    pltpu.sync_copy(idx_hbm.at[w], idxv)

    def step(i, _):
        base = base0 + i * CH
        pltpu.sync_copy(vals_hbm.at[pl.ds(base, CH)], vbuf)
        pltpu.sync_copy(vbuf, acc.at[idxv.at[i]], add=True)
        return 0
    lax.fori_loop(0, PCH, step, 0)
    plsc.subcore_barrier()

    @pl.when(sid == 0)
    def _():
        pltpu.sync_copy(acc, out_hbm.at[cid])


def _pool_scatter(vals, idx, z):
    return pl.kernel(
        _pool_body,
        out_type=jax.ShapeDtypeStruct((2, NG, GW), f32),
        mesh=_sc_mesh(),
        scratch_types=[
            pltpu.VMEM((PCH, CH), i32), pltpu.VMEM((CH, GW), f32),
            pltpu.VMEM_SHARED((NG, GW), f32),
            pltpu.SemaphoreType.DMA,
        ],
        name="sc_pool_scatter",
    )(vals, idx, z)


NB = 1000
EB = 2048


def _full(shape):
    return pl.BlockSpec(shape, lambda i: (0,) * len(shape))


def _emb_body(x_ref, w_ref, b_ref, o_ref, *, mode):
    y = jnp.dot(x_ref[...], w_ref[...], preferred_element_type=f32) + b_ref[...]
    nb, dout = y.shape
    if mode == 'pad128':
        y = jnp.concatenate([y, jnp.zeros((nb, GW - dout), f32)], axis=1)
    elif mode == 'ones128':
        y = jnp.concatenate([y, jnp.ones((nb, 8), f32),
                             jnp.zeros((nb, GW - dout - 8), f32)], axis=1)
    o_ref[...] = y


def _emb(x, w, b, mode=None):
    din, dout = w.shape
    dw = dout if mode is None else GW
    return pl.pallas_call(
        functools.partial(_emb_body, mode=mode),
        grid=(N // NB,),
        in_specs=[pl.BlockSpec((NB, din), lambda i: (i, 0)),
                  _full((din, dout)), _full((1, dout))],
        out_specs=pl.BlockSpec((NB, dw), lambda i: (i, 0)),
        out_shape=jax.ShapeDtypeStruct((N, dw), f32),
        name="tc_emb",
    )(x, w, b.reshape(1, -1))


def _prep_body(h_ref, xp_ref, wa_ref, wb_ref, b1_ref, a_ref, b_ref):
    h = h_ref[...]
    xp = xp_ref[...]
    z = jnp.zeros((h.shape[0], GW - H - 8), f32)
    ha = jnp.dot(h, wa_ref[...], preferred_element_type=f32) + b1_ref[...]
    hb = jnp.dot(h, wb_ref[...], preferred_element_type=f32)
    a_ref[...] = jnp.concatenate([ha, xp, z], axis=1)
    b_ref[...] = jnp.concatenate([hb, -xp, z], axis=1)


def _prep(h, xp, wa, wb, b1):
    return pl.pallas_call(
        _prep_body,
        grid=(N // NB,),
        in_specs=[pl.BlockSpec((NB, H), lambda i: (i, 0)),
                  pl.BlockSpec((NB, 8), lambda i: (i, 0)),
                  _full((H, H)), _full((H, H)), _full((1, H))],
        out_specs=[pl.BlockSpec((NB, GW), lambda i: (i, 0)),
                   pl.BlockSpec((NB, GW), lambda i: (i, 0))],
        out_shape=[jax.ShapeDtypeStruct((N, GW), f32),
                   jax.ShapeDtypeStruct((N, GW), f32)],
        name="tc_prep",
    )(h, xp, wa, wb, b1.reshape(1, -1))


def _edge_body(ex_ref, w1c_ref, w2_ref, b2_ref, watt_ref, batt_ref,
               wc1_ref, bc1_ref, wc2_ref, ety_ref):
    ex = ex_ref[...]
    epre = ex[:, :H]
    cd8 = ex[:, H:H + 8]
    radial = jnp.sum(cd8 * cd8, axis=1, keepdims=True)
    e1 = _silu(epre + radial * w1c_ref[...])
    e2 = _silu(jnp.dot(e1, w2_ref[...], preferred_element_type=f32)
               + b2_ref[...])
    att = jax.nn.sigmoid(jnp.sum(e2 * watt_ref[...], axis=1, keepdims=True)
                         + batt_ref[...])
    eo = e2 * att
    c1 = _silu(jnp.dot(eo, wc1_ref[...], preferred_element_type=f32)
               + bc1_ref[...])
    t = jnp.tanh(jnp.sum(c1 * wc2_ref[...], axis=1, keepdims=True))
    col8 = lax.broadcasted_iota(i32, (1, 8), 1)
    tr8 = cd8 * t + jnp.where(col8 == 3, 1.0, 0.0)
    z = jnp.zeros((eo.shape[0], GW - H - 8), f32)
    ety_ref[...] = jnp.concatenate([eo, tr8, z], axis=1)


def _edge(ex, w1c128, w2p, b2, watt, batt, wc1, bc1, wc2):
    return pl.pallas_call(
        _edge_body,
        grid=(EPAD // EB,),
        in_specs=[pl.BlockSpec((EB, GW), lambda i: (i, 0)),
                  _full((1, H)), _full((H, H)), _full((1, H)),
                  _full((1, H)), _full((1, 1)),
                  _full((H, H)), _full((1, H)), _full((1, H))],
        out_specs=pl.BlockSpec((EB, GW), lambda i: (i, 0)),
        out_shape=jax.ShapeDtypeStruct((EPAD, GW), f32),
        name="tc_edge",
    )(ex, w1c128, w2p, b2, watt, batt, wc1, bc1, wc2)


def _node_body(h_ref, xp_ref, p_ref, w1h_ref, w1a_ref, b1_ref,
               w2_ref, b2_ref, ho_ref, xo_ref):
    h = h_ref[...]
    p = p_ref[0] + p_ref[1]
    tp = p[:, H:H + 8]
    col8 = lax.broadcasted_iota(i32, (1, 8), 1)
    cnt = jnp.sum(jnp.where(col8 == 3, tp, 0.0), axis=1, keepdims=True)
    cnt = jnp.maximum(cnt, 1.0)
    xo_ref[...] = xp_ref[...] + jnp.where(col8 < 3, tp, 0.0) / cnt
    m1 = _silu(jnp.dot(h, w1h_ref[...], preferred_element_type=f32)
               + jnp.dot(p, w1a_ref[...], preferred_element_type=f32)
               + b1_ref[...])
    ho_ref[...] = h + jnp.dot(m1, w2_ref[...], preferred_element_type=f32) \
        + b2_ref[...]


def _node(h, xp, part, w1h, w1a128, b1, w2, b2):
    return pl.pallas_call(
        _node_body,
        grid=(N // NB,),
        in_specs=[pl.BlockSpec((NB, H), lambda i: (i, 0)),
                  pl.BlockSpec((NB, 8), lambda i: (i, 0)),
                  pl.BlockSpec((2, NB, GW), lambda i: (0, i, 0)),
                  _full((H, H)), _full((GW, H)), _full((1, H)),
                  _full((H, H)), _full((1, H))],
        out_specs=[pl.BlockSpec((NB, H), lambda i: (i, 0)),
                   pl.BlockSpec((NB, 8), lambda i: (i, 0))],
        out_shape=[jax.ShapeDtypeStruct((N, H), f32),
                   jax.ShapeDtypeStruct((N, 8), f32)],
        name="tc_node",
    )(h, xp, part, w1h, w1a128, b1.reshape(1, -1), w2, b2.reshape(1, -1))


def _bn_mat(g, b, m):
    mu = jnp.mean(m, axis=0, keepdims=True)
    var = jnp.mean((m - mu) ** 2, axis=0, keepdims=True)
    return g * (m - mu) * jax.lax.rsqrt(var + 1e-5) + b


def _head_body(p1_ref, p2_ref, p3_ref,
               g1_ref, b1_ref, g2_ref, b2_ref, g3_ref, b3_ref,
               wp1_ref, wp2_ref, wp3_ref, bfc_ref, gf_ref, bf_ref,
               wf_ref, bff_ref, o_ref):
    p1 = p1_ref[0] + p1_ref[1]
    p2 = p2_ref[0] + p2_ref[1]
    p3 = p3_ref[0] + p3_ref[1]
    col = lax.broadcasted_iota(i32, (1, GW), 1)
    cnt = jnp.sum(jnp.where(col == 32, p3, 0.0), axis=1, keepdims=True)
    cnt = jnp.maximum(cnt, 1.0)
    m1 = jax.nn.relu(_bn_mat(g1_ref[...], b1_ref[...], p1 / cnt))
    m2 = jax.nn.relu(_bn_mat(g2_ref[...], b2_ref[...], p2 / cnt))
    m3 = jax.nn.relu(_bn_mat(g3_ref[...], b3_ref[...], p3 / cnt))
    u = (jnp.dot(m1, wp1_ref[...], preferred_element_type=f32)
         + jnp.dot(m2, wp2_ref[...], preferred_element_type=f32)
         + jnp.dot(m3, wp3_ref[...], preferred_element_type=f32)
         + bfc_ref[...])
    u = _bn_mat(gf_ref[...], bf_ref[...], u)
    o_ref[...] = jnp.dot(u, wf_ref[...], preferred_element_type=f32) \
        + bff_ref[...]


def _head(p1, p2, p3, bn1, bn2, bn3, wp1, wp2, wp3, bfc, bnfc, wf, bff):
    df, do = 178, 128
    return pl.pallas_call(
        _head_body,
        grid=(1,),
        in_specs=[_full((2, NG, GW)), _full((2, NG, GW)), _full((2, NG, GW)),
                  _full((1, GW)), _full((1, GW)),
                  _full((1, GW)), _full((1, GW)),
                  _full((1, GW)), _full((1, GW)),
                  _full((GW, df)), _full((GW, df)), _full((GW, df)),
                  _full((1, df)), _full((1, df)), _full((1, df)),
                  _full((df, do)), _full((1, do))],
        out_specs=_full((NG, do)),
        out_shape=jax.ShapeDtypeStruct((NG, do), f32),
        name="tc_head",
    )(p1, p2, p3, bn1[0], bn1[1], bn2[0], bn2[1], bn3[0], bn3[1],
      wp1, wp2, wp3, bfc, bnfc[0], bnfc[1], wf, bff)


def _pad128(v, rows=None):
    out = jnp.zeros((rows or v.shape[0], GW), f32)
    return out.at[:v.shape[0], :v.shape[1]].set(v)


def _egnn_block(p, h_in, xp, row_g, col_g, row_s, zacc):
    h = _emb(h_in, p['emb_in']['W'], p['emb_in']['b'])
    for lp in p['layers']:
        w1 = lp['e1']['W']
        a, b = _prep(h, xp, w1[:H], w1[H:2 * H], lp['e1']['b'])
        ex = _gather_edges(a, b, row_g, col_g)
        ety = _edge(ex, w1[2 * H:2 * H + 1], lp['e2']['W'],
                    lp['e2']['b'].reshape(1, -1),
                    lp['att']['W'].reshape(1, -1),
                    lp['att']['b'].reshape(1, 1),
                    lp['c1']['W'], lp['c1']['b'].reshape(1, -1),
                    lp['c2']['W'].reshape(1, -1))
        part = _scatter_edges(ety, row_s, zacc)
        wn = lp['n1']['W']
        w1a128 = jnp.concatenate([wn[H:], jnp.zeros((GW - H, H), f32)], axis=0)
        h, xp = _node(h, xp, part, wn[:H], w1a128, lp['n1']['b'],
                      lp['n2']['W'], lp['n2']['b'])
    return h, xp


def kernel(x, coords, params, edge_index, batch):
    row, col = edge_index[0], edge_index[1]
    epad = EPAD - E
    row_g = jnp.concatenate([row, jnp.zeros((epad,), i32)]
                            ).reshape(NW, NCH, CH)
    col_g = jnp.concatenate([col, jnp.zeros((epad,), i32)]
                            ).reshape(NW, NCH, CH)
    row_s = jnp.concatenate([row, jnp.full((epad,), N + 8, i32)]
                            ).reshape(NW, NCH, CH)
    xp = jnp.pad(coords, ((0, 0), (0, 5)))
    zacc = jnp.zeros((NPS, GW), f32)
    zpool = jnp.zeros((NG, GW), f32)
    bpad = jnp.concatenate([batch, jnp.zeros((NPOOL - N,), i32)]
                           ).reshape(NW, PCH, CH)

    p = params
    h1, xp1 = _egnn_block(p['egnn1'], x, xp, row_g, col_g, row_s, zacc)
    h1 = _emb(h1, p['egnn1']['emb_out']['W'], p['egnn1']['emb_out']['b'])
    h2, xp2 = _egnn_block(p['egnn2'], h1, xp1, row_g, col_g, row_s, zacc)
    h2p = _emb(h2, p['egnn2']['emb_out']['W'], p['egnn2']['emb_out']['b'],
               mode='pad128')
    h3, _ = _egnn_block(p['egnn4'], h2p[:, :H], xp2, row_g, col_g, row_s,
                        zacc)
    h3p = _emb(h3, p['egnn4']['emb_out']['W'], p['egnn4']['emb_out']['b'],
               mode='ones128')

    v1 = jnp.zeros((NPOOL, GW), f32).at[:N].set(h1)
    v2 = jnp.zeros((NPOOL, GW), f32).at[:N].set(h2p)
    v3 = jnp.zeros((NPOOL, GW), f32).at[:N].set(h3p)
    p1 = _pool_scatter(v1, bpad, zpool)
    p2 = _pool_scatter(v2, bpad, zpool)
    p3 = _pool_scatter(v3, bpad, zpool)

    w = p['fc1']['W']
    wp1 = w[:128]
    wp2 = jnp.concatenate([w[128:192], jnp.zeros((GW - 64, 178), f32)])
    wp3 = jnp.concatenate([w[192:224], jnp.zeros((GW - 32, 178), f32)])
    out = _head(p1, p2, p3,
                (_pad128(p['bn1']['g'].reshape(1, -1), 1),
                 _pad128(p['bn1']['b'].reshape(1, -1), 1)),
                (_pad128(p['bn2']['g'].reshape(1, -1), 1),
                 _pad128(p['bn2']['b'].reshape(1, -1), 1)),
                (_pad128(p['bn3']['g'].reshape(1, -1), 1),
                 _pad128(p['bn3']['b'].reshape(1, -1), 1)),
                wp1, wp2, wp3, p['fc1']['b'].reshape(1, -1),
                (p['bn_fc1']['g'].reshape(1, -1),
                 p['bn_fc1']['b'].reshape(1, -1)),
                p['final']['W'], p['final']['b'].reshape(1, -1))
    return out


# speedup vs baseline: 1.0467x; 1.0318x over previous
import functools
import jax
import jax.numpy as jnp
from jax import lax
from jax.experimental import pallas as pl
from jax.experimental.pallas import tpu as pltpu
from jax.experimental.pallas import tpu_sc as plsc

f32 = jnp.float32
i32 = jnp.int32

N = 10000
E = 320000
NG = 64
H = 64
GW = 128
NW = 32
CH = 128
EPAD = NW * 80 * CH
NCH = 80
NACC = 10240
NPS = NACC // 16
NPOOL = 12288
PCH = 3


@functools.cache
def _sc_mesh():
    return plsc.VectorSubcoreMesh(core_axis_name="c", subcore_axis_name="s",
                                  num_cores=2, num_subcores=16)


def _silu(x):
    return x * jax.nn.sigmoid(x)


def _gather_body(a_hbm, b_hbm, row_hbm, col_hbm, out_hbm,
                 rowi, coli, bufa0, bufb0, bufa1, bufb1,
                 sem0, sem1, osem0, osem1):
    cid = lax.axis_index("c")
    sid = lax.axis_index("s")
    w = sid * 2 + cid
    base0 = w * (NCH * CH)

    pltpu.sync_copy(row_hbm.at[w], rowi)
    pltpu.sync_copy(col_hbm.at[w], coli)

    sets = ((bufa0, bufb0, sem0, osem0),
            (bufa1, bufb1, sem1, osem1))

    def stage(i, s):
        bufa, bufb, sem, osem = s
        pltpu.async_copy(a_hbm.at[rowi.at[i]], bufa, sem)
        pltpu.async_copy(b_hbm.at[coli.at[i]], bufb, sem)

    def process(i, s):
        bufa, bufb, sem, osem = s
        base = base0 + i * CH
        pltpu.make_async_copy(a_hbm.at[rowi.at[i]], bufa, sem).wait()
        pltpu.make_async_copy(b_hbm.at[coli.at[i]], bufb, sem).wait()

        @pl.loop(0, CH, unroll=8)
        def _(r):
            for c in range(5):
                bufa[r, pl.ds(c * 16, 16)] = (bufa[r, pl.ds(c * 16, 16)] +
                                              bufb[r, pl.ds(c * 16, 16)])
        pltpu.async_copy(bufa, out_hbm.at[pl.ds(base, CH)], osem)

    def waitout(i, s):
        bufa, bufb, sem, osem = s
        base = base0 + i * CH
        pltpu.make_async_copy(bufa, out_hbm.at[pl.ds(base, CH)], osem).wait()

    stage(0, sets[0])

    @pl.loop(0, NCH, step=2)
    def _(i):
        @pl.when(i + 1 < NCH)
        def _():
            stage(i + 1, sets[1])
        process(i, sets[0])

        @pl.when(i + 2 < NCH)
        def _():
            waitout(i, sets[0])
            stage(i + 2, sets[0])

        @pl.when(i + 1 < NCH)
        def _():
            process(i + 1, sets[1])

        @pl.when(i + 3 < NCH)
        def _():
            waitout(i + 1, sets[1])

    waitout(NCH - 2, sets[0])
    waitout(NCH - 1, sets[1])


def _gather_edges(a, b, row3, col3):
    return pl.kernel(
        _gather_body,
        out_type=jax.ShapeDtypeStruct((EPAD, GW), f32),
        mesh=_sc_mesh(),
        scratch_types=[
            pltpu.VMEM((NCH, CH), i32), pltpu.VMEM((NCH, CH), i32),
            pltpu.VMEM((CH, GW), f32), pltpu.VMEM((CH, GW), f32),
            pltpu.VMEM((CH, GW), f32), pltpu.VMEM((CH, GW), f32),
            pltpu.SemaphoreType.DMA, pltpu.SemaphoreType.DMA,
            pltpu.SemaphoreType.DMA, pltpu.SemaphoreType.DMA,
        ],
        name="sc_gather_edges",
    )(a, b, row3, col3)


def _scatter_body(ety_hbm, row_hbm, z_hbm, part_hbm,
                  rowi, vbuf0, vbuf1,
                  acc, sem0, sem1, ssem0, ssem1):
    cid = lax.axis_index("c")
    sid = lax.axis_index("s")
    w = sid * 2 + cid
    base0 = w * (NCH * CH)

    pltpu.sync_copy(row_hbm.at[w], rowi)
    pltpu.sync_copy(z_hbm, acc.at[pl.ds(sid * NPS, NPS)])
    plsc.subcore_barrier()

    sets = ((vbuf0, sem0, ssem0), (vbuf1, sem1, ssem1))

    def stage(i, s):
        vbuf, sem, ssem = s
        base = base0 + i * CH
        pltpu.async_copy(ety_hbm.at[pl.ds(base, CH)], vbuf, sem)

    def process(i, s):
        vbuf, sem, ssem = s
        base = base0 + i * CH
        pltpu.make_async_copy(ety_hbm.at[pl.ds(base, CH)], vbuf, sem).wait()
        pltpu.async_copy(vbuf, acc.at[rowi.at[i]], ssem, add=True)

    def waitsc(i, s):
        vbuf, sem, ssem = s
        pltpu.make_async_copy(vbuf, acc.at[rowi.at[i]], ssem).wait()

    stage(0, sets[0])

    @pl.loop(0, NCH, step=2)
    def _(i):
        @pl.when(i + 1 < NCH)
        def _():
            stage(i + 1, sets[1])
        process(i, sets[0])

        @pl.when(i + 2 < NCH)
        def _():
            waitsc(i, sets[0])
            stage(i + 2, sets[0])

        @pl.when(i + 1 < NCH)
        def _():
            process(i + 1, sets[1])

        @pl.when(i + 3 < NCH)
        def _():
            waitsc(i + 1, sets[1])

    waitsc(NCH - 2, sets[0])
    waitsc(NCH - 1, sets[1])
    plsc.subcore_barrier()

    pltpu.sync_copy(acc.at[pl.ds(sid * NPS, NPS)],
                    part_hbm.at[cid, pl.ds(sid * NPS, NPS)])


def _scatter_edges(ety, row3, z):
    return pl.kernel(
        _scatter_body,
        out_type=jax.ShapeDtypeStruct((2, NACC, GW), f32),
        mesh=_sc_mesh(),
        scratch_types=[
            pltpu.VMEM((NCH, CH), i32),
            pltpu.VMEM((CH, GW), f32), pltpu.VMEM((CH, GW), f32),
            pltpu.VMEM_SHARED((NACC, GW), f32),
            pltpu.SemaphoreType.DMA, pltpu.SemaphoreType.DMA,
            pltpu.SemaphoreType.DMA, pltpu.SemaphoreType.DMA,
        ],
        name="sc_scatter_edges",
    )(ety, row3, z)


def _pool_body(vals_hbm, idx_hbm, z_hbm, out_hbm, idxv, vbuf, acc, sem):
    cid = lax.axis_index("c")
    sid = lax.axis_index("s")
    w = sid * 2 + cid
    base0 = w * (PCH * CH)

    @pl.when(sid == 0)
    def _():
        pltpu.sync_copy(z_hbm, acc)
    plsc.subcore_barrier()

    pltpu.sync_copy(idx_hbm.at[w], idxv)

    def step(i, _):
        base = base0 + i * CH
        pltpu.sync_copy(vals_hbm.at[pl.ds(base, CH)], vbuf)
        pltpu.sync_copy(vbuf, acc.at[idxv.at[i]], add=True)
        return 0
    lax.fori_loop(0, PCH, step, 0)
    plsc.subcore_barrier()

    @pl.when(sid == 0)
    def _():
        pltpu.sync_copy(acc, out_hbm.at[cid])


def _pool_scatter(vals, idx, z):
    return pl.kernel(
        _pool_body,
        out_type=jax.ShapeDtypeStruct((2, NG, GW), f32),
        mesh=_sc_mesh(),
        scratch_types=[
            pltpu.VMEM((PCH, CH), i32), pltpu.VMEM((CH, GW), f32),
            pltpu.VMEM_SHARED((NG, GW), f32),
            pltpu.SemaphoreType.DMA,
        ],
        name="sc_pool_scatter",
    )(vals, idx, z)


NB = 1000
EB = 2048


def _full(shape):
    return pl.BlockSpec(shape, lambda i: (0,) * len(shape))


def _emb_body(x_ref, w_ref, b_ref, o_ref, *, mode):
    y = jnp.dot(x_ref[...], w_ref[...], preferred_element_type=f32) + b_ref[...]
    nb, dout = y.shape
    if mode == 'pad128':
        y = jnp.concatenate([y, jnp.zeros((nb, GW - dout), f32)], axis=1)
    elif mode == 'ones128':
        y = jnp.concatenate([y, jnp.ones((nb, 8), f32),
                             jnp.zeros((nb, GW - dout - 8), f32)], axis=1)
    o_ref[...] = y


def _emb(x, w, b, mode=None):
    din, dout = w.shape
    dw = dout if mode is None else GW
    return pl.pallas_call(
        functools.partial(_emb_body, mode=mode),
        grid=(N // NB,),
        in_specs=[pl.BlockSpec((NB, din), lambda i: (i, 0)),
                  _full((din, dout)), _full((1, dout))],
        out_specs=pl.BlockSpec((NB, dw), lambda i: (i, 0)),
        out_shape=jax.ShapeDtypeStruct((N, dw), f32),
        name="tc_emb",
    )(x, w, b.reshape(1, -1))


def _prep_body(h_ref, xp_ref, wa_ref, wb_ref, b1_ref, a_ref, b_ref):
    h = h_ref[...]
    xp = xp_ref[...]
    z = jnp.zeros((h.shape[0], GW - H - 8), f32)
    ha = jnp.dot(h, wa_ref[...], preferred_element_type=f32) + b1_ref[...]
    hb = jnp.dot(h, wb_ref[...], preferred_element_type=f32)
    a_ref[...] = jnp.concatenate([ha, xp, z], axis=1)
    b_ref[...] = jnp.concatenate([hb, -xp, z], axis=1)


def _prep(h, xp, wa, wb, b1):
    return pl.pallas_call(
        _prep_body,
        grid=(N // NB,),
        in_specs=[pl.BlockSpec((NB, H), lambda i: (i, 0)),
                  pl.BlockSpec((NB, 8), lambda i: (i, 0)),
                  _full((H, H)), _full((H, H)), _full((1, H))],
        out_specs=[pl.BlockSpec((NB, GW), lambda i: (i, 0)),
                   pl.BlockSpec((NB, GW), lambda i: (i, 0))],
        out_shape=[jax.ShapeDtypeStruct((N, GW), f32),
                   jax.ShapeDtypeStruct((N, GW), f32)],
        name="tc_prep",
    )(h, xp, wa, wb, b1.reshape(1, -1))


def _edge_body(ex_ref, w1c_ref, w2_ref, b2_ref, watt_ref, batt_ref,
               wc1_ref, bc1_ref, wc2_ref, ety_ref):
    ex = ex_ref[...]
    col = lax.broadcasted_iota(i32, (1, GW), 1)
    exq = jnp.where(col >= H, ex * ex, 0.0)
    radial = jnp.sum(exq, axis=1, keepdims=True)
    e1 = _silu(ex + radial * w1c_ref[...])
    e2 = _silu(jnp.dot(e1, w2_ref[...], preferred_element_type=f32)
               + b2_ref[...])
    cd8 = ex[:, H:H + 8]
    att = jax.nn.sigmoid(jnp.sum(e2 * watt_ref[...], axis=1, keepdims=True)
                         + batt_ref[...])
    eo = e2 * att
    c1 = _silu(jnp.dot(eo, wc1_ref[...], preferred_element_type=f32)
               + bc1_ref[...])
    t = jnp.tanh(jnp.sum(c1 * wc2_ref[...], axis=1, keepdims=True))
    col8 = lax.broadcasted_iota(i32, (1, 8), 1)
    tr8 = cd8 * t + jnp.where(col8 == 3, 1.0, 0.0)
    z = jnp.zeros((eo.shape[0], GW - H - 8), f32)
    ety_ref[...] = jnp.concatenate([eo, tr8, z], axis=1)


def _edge(ex, w1c128, w2p, b2, watt, batt, wc1, bc1, wc2):
    return pl.pallas_call(
        _edge_body,
        grid=(EPAD // EB,),
        in_specs=[pl.BlockSpec((EB, GW), lambda i: (i, 0)),
                  _full((1, GW)), _full((GW, H)), _full((1, H)),
                  _full((1, H)), _full((1, 1)),
                  _full((H, H)), _full((1, H)), _full((1, H))],
        out_specs=pl.BlockSpec((EB, GW), lambda i: (i, 0)),
        out_shape=jax.ShapeDtypeStruct((EPAD, GW), f32),
        name="tc_edge",
    )(ex, w1c128, w2p, b2, watt, batt, wc1, bc1, wc2)


def _node_body(h_ref, xp_ref, p_ref, w1h_ref, w1a_ref, b1_ref,
               w2_ref, b2_ref, ho_ref, xo_ref):
    h = h_ref[...]
    p = p_ref[0] + p_ref[1]
    tp = p[:, H:H + 8]
    col8 = lax.broadcasted_iota(i32, (1, 8), 1)
    cnt = jnp.sum(jnp.where(col8 == 3, tp, 0.0), axis=1, keepdims=True)
    cnt = jnp.maximum(cnt, 1.0)
    xo_ref[...] = xp_ref[...] + jnp.where(col8 < 3, tp, 0.0) / cnt
    m1 = _silu(jnp.dot(h, w1h_ref[...], preferred_element_type=f32)
               + jnp.dot(p, w1a_ref[...], preferred_element_type=f32)
               + b1_ref[...])
    ho_ref[...] = h + jnp.dot(m1, w2_ref[...], preferred_element_type=f32) \
        + b2_ref[...]


def _node(h, xp, part, w1h, w1a128, b1, w2, b2):
    return pl.pallas_call(
        _node_body,
        grid=(N // NB,),
        in_specs=[pl.BlockSpec((NB, H), lambda i: (i, 0)),
                  pl.BlockSpec((NB, 8), lambda i: (i, 0)),
                  pl.BlockSpec((2, NB, GW), lambda i: (0, i, 0)),
                  _full((H, H)), _full((GW, H)), _full((1, H)),
                  _full((H, H)), _full((1, H))],
        out_specs=[pl.BlockSpec((NB, H), lambda i: (i, 0)),
                   pl.BlockSpec((NB, 8), lambda i: (i, 0))],
        out_shape=[jax.ShapeDtypeStruct((N, H), f32),
                   jax.ShapeDtypeStruct((N, 8), f32)],
        name="tc_node",
    )(h, xp, part, w1h, w1a128, b1.reshape(1, -1), w2, b2.reshape(1, -1))


def _bn_mat(g, b, m):
    mu = jnp.mean(m, axis=0, keepdims=True)
    var = jnp.mean((m - mu) ** 2, axis=0, keepdims=True)
    return g * (m - mu) * jax.lax.rsqrt(var + 1e-5) + b


def _head_body(p1_ref, p2_ref, p3_ref,
               g1_ref, b1_ref, g2_ref, b2_ref, g3_ref, b3_ref,
               wp1_ref, wp2_ref, wp3_ref, bfc_ref, gf_ref, bf_ref,
               wf_ref, bff_ref, o_ref):
    p1 = p1_ref[0] + p1_ref[1]
    p2 = p2_ref[0] + p2_ref[1]
    p3 = p3_ref[0] + p3_ref[1]
    col = lax.broadcasted_iota(i32, (1, GW), 1)
    cnt = jnp.sum(jnp.where(col == 32, p3, 0.0), axis=1, keepdims=True)
    cnt = jnp.maximum(cnt, 1.0)
    m1 = jax.nn.relu(_bn_mat(g1_ref[...], b1_ref[...], p1 / cnt))
    m2 = jax.nn.relu(_bn_mat(g2_ref[...], b2_ref[...], p2 / cnt))
    m3 = jax.nn.relu(_bn_mat(g3_ref[...], b3_ref[...], p3 / cnt))
    u = (jnp.dot(m1, wp1_ref[...], preferred_element_type=f32)
         + jnp.dot(m2, wp2_ref[...], preferred_element_type=f32)
         + jnp.dot(m3, wp3_ref[...], preferred_element_type=f32)
         + bfc_ref[...])
    u = _bn_mat(gf_ref[...], bf_ref[...], u)
    o_ref[...] = jnp.dot(u, wf_ref[...], preferred_element_type=f32) \
        + bff_ref[...]


def _head(p1, p2, p3, bn1, bn2, bn3, wp1, wp2, wp3, bfc, bnfc, wf, bff):
    df, do = 178, 128
    return pl.pallas_call(
        _head_body,
        grid=(1,),
        in_specs=[_full((2, NG, GW)), _full((2, NG, GW)), _full((2, NG, GW)),
                  _full((1, GW)), _full((1, GW)),
                  _full((1, GW)), _full((1, GW)),
                  _full((1, GW)), _full((1, GW)),
                  _full((GW, df)), _full((GW, df)), _full((GW, df)),
                  _full((1, df)), _full((1, df)), _full((1, df)),
                  _full((df, do)), _full((1, do))],
        out_specs=_full((NG, do)),
        out_shape=jax.ShapeDtypeStruct((NG, do), f32),
        name="tc_head",
    )(p1, p2, p3, bn1[0], bn1[1], bn2[0], bn2[1], bn3[0], bn3[1],
      wp1, wp2, wp3, bfc, bnfc[0], bnfc[1], wf, bff)


def _pad128(v, rows=None):
    out = jnp.zeros((rows or v.shape[0], GW), f32)
    return out.at[:v.shape[0], :v.shape[1]].set(v)


def _egnn_block(p, h_in, xp, row_g, col_g, row_s, zacc):
    h = _emb(h_in, p['emb_in']['W'], p['emb_in']['b'])
    for lp in p['layers']:
        w1 = lp['e1']['W']
        a, b = _prep(h, xp, w1[:H], w1[H:2 * H], lp['e1']['b'])
        ex = _gather_edges(a, b, row_g, col_g)
        w1c128 = _pad128(w1[2 * H:2 * H + 1], 1)
        w2p = jnp.concatenate([lp['e2']['W'],
                               jnp.zeros((GW - H, H), f32)], axis=0)
        ety = _edge(ex, w1c128, w2p, lp['e2']['b'].reshape(1, -1),
                    lp['att']['W'].reshape(1, -1),
                    lp['att']['b'].reshape(1, 1),
                    lp['c1']['W'], lp['c1']['b'].reshape(1, -1),
                    lp['c2']['W'].reshape(1, -1))
        part = _scatter_edges(ety, row_s, zacc)
        wn = lp['n1']['W']
        w1a128 = jnp.concatenate([wn[H:], jnp.zeros((GW - H, H), f32)], axis=0)
        h, xp = _node(h, xp, part, wn[:H], w1a128, lp['n1']['b'],
                      lp['n2']['W'], lp['n2']['b'])
    return h, xp


def kernel(x, coords, params, edge_index, batch):
    row, col = edge_index[0], edge_index[1]
    epad = EPAD - E
    row_g = jnp.concatenate([row, jnp.zeros((epad,), i32)]
                            ).reshape(NW, NCH, CH)
    col_g = jnp.concatenate([col, jnp.zeros((epad,), i32)]
                            ).reshape(NW, NCH, CH)
    row_s = jnp.concatenate([row, jnp.full((epad,), N + 8, i32)]
                            ).reshape(NW, NCH, CH)
    xp = jnp.pad(coords, ((0, 0), (0, 5)))
    zacc = jnp.zeros((NPS, GW), f32)
    zpool = jnp.zeros((NG, GW), f32)
    bpad = jnp.concatenate([batch, jnp.zeros((NPOOL - N,), i32)]
                           ).reshape(NW, PCH, CH)

    p = params
    h1, xp1 = _egnn_block(p['egnn1'], x, xp, row_g, col_g, row_s, zacc)
    h1 = _emb(h1, p['egnn1']['emb_out']['W'], p['egnn1']['emb_out']['b'])
    h2, xp2 = _egnn_block(p['egnn2'], h1, xp1, row_g, col_g, row_s, zacc)
    h2p = _emb(h2, p['egnn2']['emb_out']['W'], p['egnn2']['emb_out']['b'],
               mode='pad128')
    h3, _ = _egnn_block(p['egnn4'], h2p[:, :H], xp2, row_g, col_g, row_s,
                        zacc)
    h3p = _emb(h3, p['egnn4']['emb_out']['W'], p['egnn4']['emb_out']['b'],
               mode='ones128')

    v1 = jnp.zeros((NPOOL, GW), f32).at[:N].set(h1)
    v2 = jnp.zeros((NPOOL, GW), f32).at[:N].set(h2p)
    v3 = jnp.zeros((NPOOL, GW), f32).at[:N].set(h3p)
    p1 = _pool_scatter(v1, bpad, zpool)
    p2 = _pool_scatter(v2, bpad, zpool)
    p3 = _pool_scatter(v3, bpad, zpool)

    w = p['fc1']['W']
    wp1 = w[:128]
    wp2 = jnp.concatenate([w[128:192], jnp.zeros((GW - 64, 178), f32)])
    wp3 = jnp.concatenate([w[192:224], jnp.zeros((GW - 32, 178), f32)])
    out = _head(p1, p2, p3,
                (_pad128(p['bn1']['g'].reshape(1, -1), 1),
                 _pad128(p['bn1']['b'].reshape(1, -1), 1)),
                (_pad128(p['bn2']['g'].reshape(1, -1), 1),
                 _pad128(p['bn2']['b'].reshape(1, -1), 1)),
                (_pad128(p['bn3']['g'].reshape(1, -1), 1),
                 _pad128(p['bn3']['b'].reshape(1, -1), 1)),
                wp1, wp2, wp3, p['fc1']['b'].reshape(1, -1),
                (p['bn_fc1']['g'].reshape(1, -1),
                 p['bn_fc1']['b'].reshape(1, -1)),
                p['final']['W'], p['final']['b'].reshape(1, -1))
    return out


# speedup vs baseline: 1.1004x; 1.0513x over previous
import functools
import jax
import jax.numpy as jnp
from jax import lax
from jax.experimental import pallas as pl
from jax.experimental.pallas import tpu as pltpu
from jax.experimental.pallas import tpu_sc as plsc

f32 = jnp.float32
i32 = jnp.int32

N = 10000
E = 320000
NG = 64
H = 64
GW = 128
NW = 32
CH = 128
EPAD = NW * 80 * CH
NCH = 80
NACC = 10240
NPS = NACC // 16
NPOOL = 12288
PCH = 3


@functools.cache
def _sc_mesh():
    return plsc.VectorSubcoreMesh(core_axis_name="c", subcore_axis_name="s",
                                  num_cores=2, num_subcores=16)


def _silu(x):
    return x * jax.nn.sigmoid(x)


def _gather_body(a_hbm, b_hbm, row_hbm, col_hbm, out_hbm,
                 rowi, coli, bufa0, bufb0, bufa1, bufb1,
                 sem0, sem1, osem0, osem1):
    cid = lax.axis_index("c")
    sid = lax.axis_index("s")
    w = sid * 2 + cid
    base0 = w * (NCH * CH)

    pltpu.sync_copy(row_hbm.at[w], rowi)
    pltpu.sync_copy(col_hbm.at[w], coli)

    sets = ((bufa0, bufb0, sem0, osem0),
            (bufa1, bufb1, sem1, osem1))

    def stage(i, s):
        bufa, bufb, sem, osem = s
        pltpu.async_copy(a_hbm.at[rowi.at[i]], bufa, sem)
        pltpu.async_copy(b_hbm.at[coli.at[i]], bufb, sem)

    def process(i, s):
        bufa, bufb, sem, osem = s
        base = base0 + i * CH
        pltpu.make_async_copy(a_hbm.at[rowi.at[i]], bufa, sem).wait()
        pltpu.make_async_copy(b_hbm.at[coli.at[i]], bufb, sem).wait()

        def add_row(r, _):
            for c in range(5):
                bufa[r, pl.ds(c * 16, 16)] = (bufa[r, pl.ds(c * 16, 16)] +
                                              bufb[r, pl.ds(c * 16, 16)])
            return 0
        lax.fori_loop(0, CH, add_row, 0)
        pltpu.async_copy(bufa, out_hbm.at[pl.ds(base, CH)], osem)

    def waitout(i, s):
        bufa, bufb, sem, osem = s
        base = base0 + i * CH
        pltpu.make_async_copy(bufa, out_hbm.at[pl.ds(base, CH)], osem).wait()

    stage(0, sets[0])

    @pl.loop(0, NCH, step=2)
    def _(i):
        @pl.when(i + 1 < NCH)
        def _():
            stage(i + 1, sets[1])
        process(i, sets[0])

        @pl.when(i + 2 < NCH)
        def _():
            waitout(i, sets[0])
            stage(i + 2, sets[0])

        @pl.when(i + 1 < NCH)
        def _():
            process(i + 1, sets[1])

        @pl.when(i + 3 < NCH)
        def _():
            waitout(i + 1, sets[1])

    waitout(NCH - 2, sets[0])
    waitout(NCH - 1, sets[1])


def _gather_edges(a, b, row3, col3):
    return pl.kernel(
        _gather_body,
        out_type=jax.ShapeDtypeStruct((EPAD, GW), f32),
        mesh=_sc_mesh(),
        scratch_types=[
            pltpu.VMEM((NCH, CH), i32), pltpu.VMEM((NCH, CH), i32),
            pltpu.VMEM((CH, GW), f32), pltpu.VMEM((CH, GW), f32),
            pltpu.VMEM((CH, GW), f32), pltpu.VMEM((CH, GW), f32),
            pltpu.SemaphoreType.DMA, pltpu.SemaphoreType.DMA,
            pltpu.SemaphoreType.DMA, pltpu.SemaphoreType.DMA,
        ],
        name="sc_gather_edges",
    )(a, b, row3, col3)


def _scatter_body(ety_hbm, row_hbm, z_hbm, part_hbm,
                  rowi, vbuf0, vbuf1,
                  acc, sem0, sem1, ssem0, ssem1):
    cid = lax.axis_index("c")
    sid = lax.axis_index("s")
    w = sid * 2 + cid
    base0 = w * (NCH * CH)

    pltpu.sync_copy(row_hbm.at[w], rowi)
    pltpu.sync_copy(z_hbm, acc.at[pl.ds(sid * NPS, NPS)])
    plsc.subcore_barrier()

    sets = ((vbuf0, sem0, ssem0), (vbuf1, sem1, ssem1))

    def stage(i, s):
        vbuf, sem, ssem = s
        base = base0 + i * CH
        pltpu.async_copy(ety_hbm.at[pl.ds(base, CH)], vbuf, sem)

    def process(i, s):
        vbuf, sem, ssem = s
        base = base0 + i * CH
        pltpu.make_async_copy(ety_hbm.at[pl.ds(base, CH)], vbuf, sem).wait()
        pltpu.sync_copy(vbuf, acc.at[rowi.at[i]], add=True)

    def waitsc(i, s):
        pass

    stage(0, sets[0])

    @pl.loop(0, NCH, step=2)
    def _(i):
        @pl.when(i + 1 < NCH)
        def _():
            stage(i + 1, sets[1])
        process(i, sets[0])

        @pl.when(i + 2 < NCH)
        def _():
            waitsc(i, sets[0])
            stage(i + 2, sets[0])

        @pl.when(i + 1 < NCH)
        def _():
            process(i + 1, sets[1])

        @pl.when(i + 3 < NCH)
        def _():
            waitsc(i + 1, sets[1])

    waitsc(NCH - 2, sets[0])
    waitsc(NCH - 1, sets[1])
    plsc.subcore_barrier()

    pltpu.sync_copy(acc.at[pl.ds(sid * NPS, NPS)],
                    part_hbm.at[cid, pl.ds(sid * NPS, NPS)])


def _scatter_edges(ety, row3, z):
    return pl.kernel(
        _scatter_body,
        out_type=jax.ShapeDtypeStruct((2, NACC, GW), f32),
        mesh=_sc_mesh(),
        scratch_types=[
            pltpu.VMEM((NCH, CH), i32),
            pltpu.VMEM((CH, GW), f32), pltpu.VMEM((CH, GW), f32),
            pltpu.VMEM_SHARED((NACC, GW), f32),
            pltpu.SemaphoreType.DMA, pltpu.SemaphoreType.DMA,
            pltpu.SemaphoreType.DMA, pltpu.SemaphoreType.DMA,
        ],
        name="sc_scatter_edges",
    )(ety, row3, z)


def _pool_body(vals_hbm, idx_hbm, z_hbm, out_hbm, idxv, vbuf, acc, sem):
    cid = lax.axis_index("c")
    sid = lax.axis_index("s")
    w = sid * 2 + cid
    base0 = w * (PCH * CH)

    @pl.when(sid == 0)
    def _():
        pltpu.sync_copy(z_hbm, acc)
    plsc.subcore_barrier()

    pltpu.sync_copy(idx_hbm.at[w], idxv)

    def step(i, _):
        base = base0 + i * CH
        pltpu.sync_copy(vals_hbm.at[pl.ds(base, CH)], vbuf)
        pltpu.sync_copy(vbuf, acc.at[idxv.at[i]], add=True)
        return 0
    lax.fori_loop(0, PCH, step, 0)
    plsc.subcore_barrier()

    @pl.when(sid == 0)
    def _():
        pltpu.sync_copy(acc, out_hbm.at[cid])


def _pool_scatter(vals, idx, z):
    return pl.kernel(
        _pool_body,
        out_type=jax.ShapeDtypeStruct((2, NG, GW), f32),
        mesh=_sc_mesh(),
        scratch_types=[
            pltpu.VMEM((PCH, CH), i32), pltpu.VMEM((CH, GW), f32),
            pltpu.VMEM_SHARED((NG, GW), f32),
            pltpu.SemaphoreType.DMA,
        ],
        name="sc_pool_scatter",
    )(vals, idx, z)


NB = 1000
EB = 2048


def _full(shape):
    return pl.BlockSpec(shape, lambda i: (0,) * len(shape))


def _emb_body(x_ref, w_ref, b_ref, o_ref, *, mode):
    y = jnp.dot(x_ref[...], w_ref[...], preferred_element_type=f32) + b_ref[...]
    nb, dout = y.shape
    if mode == 'pad128':
        y = jnp.concatenate([y, jnp.zeros((nb, GW - dout), f32)], axis=1)
    elif mode == 'ones128':
        y = jnp.concatenate([y, jnp.ones((nb, 8), f32),
                             jnp.zeros((nb, GW - dout - 8), f32)], axis=1)
    o_ref[...] = y


def _emb(x, w, b, mode=None):
    din, dout = w.shape
    dw = dout if mode is None else GW
    return pl.pallas_call(
        functools.partial(_emb_body, mode=mode),
        grid=(N // NB,),
        in_specs=[pl.BlockSpec((NB, din), lambda i: (i, 0)),
                  _full((din, dout)), _full((1, dout))],
        out_specs=pl.BlockSpec((NB, dw), lambda i: (i, 0)),
        out_shape=jax.ShapeDtypeStruct((N, dw), f32),
        name="tc_emb",
    )(x, w, b.reshape(1, -1))


def _prep_body(h_ref, xp_ref, wa_ref, wb_ref, b1_ref, a_ref, b_ref):
    h = h_ref[...]
    xp = xp_ref[...]
    z = jnp.zeros((h.shape[0], GW - H - 8), f32)
    ha = jnp.dot(h, wa_ref[...], preferred_element_type=f32) + b1_ref[...]
    hb = jnp.dot(h, wb_ref[...], preferred_element_type=f32)
    a_ref[...] = jnp.concatenate([ha, xp, z], axis=1)
    b_ref[...] = jnp.concatenate([hb, -xp, z], axis=1)


def _prep(h, xp, wa, wb, b1):
    return pl.pallas_call(
        _prep_body,
        grid=(N // NB,),
        in_specs=[pl.BlockSpec((NB, H), lambda i: (i, 0)),
                  pl.BlockSpec((NB, 8), lambda i: (i, 0)),
                  _full((H, H)), _full((H, H)), _full((1, H))],
        out_specs=[pl.BlockSpec((NB, GW), lambda i: (i, 0)),
                   pl.BlockSpec((NB, GW), lambda i: (i, 0))],
        out_shape=[jax.ShapeDtypeStruct((N, GW), f32),
                   jax.ShapeDtypeStruct((N, GW), f32)],
        name="tc_prep",
    )(h, xp, wa, wb, b1.reshape(1, -1))


def _edge_body(ex_ref, w1c_ref, w2_ref, b2_ref, watt_ref, batt_ref,
               wc1_ref, bc1_ref, wc2_ref, ety_ref):
    ex = ex_ref[...]
    col = lax.broadcasted_iota(i32, (1, GW), 1)
    exq = jnp.where(col >= H, ex * ex, 0.0)
    radial = jnp.sum(exq, axis=1, keepdims=True)
    e1 = _silu(ex + radial * w1c_ref[...])
    e2 = _silu(jnp.dot(e1, w2_ref[...], preferred_element_type=f32)
               + b2_ref[...])
    cd8 = ex[:, H:H + 8]
    att = jax.nn.sigmoid(jnp.sum(e2 * watt_ref[...], axis=1, keepdims=True)
                         + batt_ref[...])
    eo = e2 * att
    c1 = _silu(jnp.dot(eo, wc1_ref[...], preferred_element_type=f32)
               + bc1_ref[...])
    t = jnp.tanh(jnp.sum(c1 * wc2_ref[...], axis=1, keepdims=True))
    col8 = lax.broadcasted_iota(i32, (1, 8), 1)
    tr8 = cd8 * t + jnp.where(col8 == 3, 1.0, 0.0)
    z = jnp.zeros((eo.shape[0], GW - H - 8), f32)
    ety_ref[...] = jnp.concatenate([eo, tr8, z], axis=1)


def _edge(ex, w1c128, w2p, b2, watt, batt, wc1, bc1, wc2):
    return pl.pallas_call(
        _edge_body,
        grid=(EPAD // EB,),
        in_specs=[pl.BlockSpec((EB, GW), lambda i: (i, 0)),
                  _full((1, GW)), _full((GW, H)), _full((1, H)),
                  _full((1, H)), _full((1, 1)),
                  _full((H, H)), _full((1, H)), _full((1, H))],
        out_specs=pl.BlockSpec((EB, GW), lambda i: (i, 0)),
        out_shape=jax.ShapeDtypeStruct((EPAD, GW), f32),
        name="tc_edge",
    )(ex, w1c128, w2p, b2, watt, batt, wc1, bc1, wc2)


def _node_body(h_ref, xp_ref, p_ref, w1h_ref, w1a_ref, b1_ref,
               w2_ref, b2_ref, ho_ref, xo_ref):
    h = h_ref[...]
    p = p_ref[0] + p_ref[1]
    tp = p[:, H:H + 8]
    col8 = lax.broadcasted_iota(i32, (1, 8), 1)
    cnt = jnp.sum(jnp.where(col8 == 3, tp, 0.0), axis=1, keepdims=True)
    cnt = jnp.maximum(cnt, 1.0)
    xo_ref[...] = xp_ref[...] + jnp.where(col8 < 3, tp, 0.0) / cnt
    m1 = _silu(jnp.dot(h, w1h_ref[...], preferred_element_type=f32)
               + jnp.dot(p, w1a_ref[...], preferred_element_type=f32)
               + b1_ref[...])
    ho_ref[...] = h + jnp.dot(m1, w2_ref[...], preferred_element_type=f32) \
        + b2_ref[...]


def _node(h, xp, part, w1h, w1a128, b1, w2, b2):
    return pl.pallas_call(
        _node_body,
        grid=(N // NB,),
        in_specs=[pl.BlockSpec((NB, H), lambda i: (i, 0)),
                  pl.BlockSpec((NB, 8), lambda i: (i, 0)),
                  pl.BlockSpec((2, NB, GW), lambda i: (0, i, 0)),
                  _full((H, H)), _full((GW, H)), _full((1, H)),
                  _full((H, H)), _full((1, H))],
        out_specs=[pl.BlockSpec((NB, H), lambda i: (i, 0)),
                   pl.BlockSpec((NB, 8), lambda i: (i, 0))],
        out_shape=[jax.ShapeDtypeStruct((N, H), f32),
                   jax.ShapeDtypeStruct((N, 8), f32)],
        name="tc_node",
    )(h, xp, part, w1h, w1a128, b1.reshape(1, -1), w2, b2.reshape(1, -1))


def _bn_mat(g, b, m):
    mu = jnp.mean(m, axis=0, keepdims=True)
    var = jnp.mean((m - mu) ** 2, axis=0, keepdims=True)
    return g * (m - mu) * jax.lax.rsqrt(var + 1e-5) + b


def _head_body(p1_ref, p2_ref, p3_ref,
               g1_ref, b1_ref, g2_ref, b2_ref, g3_ref, b3_ref,
               wp1_ref, wp2_ref, wp3_ref, bfc_ref, gf_ref, bf_ref,
               wf_ref, bff_ref, o_ref):
    p1 = p1_ref[0] + p1_ref[1]
    p2 = p2_ref[0] + p2_ref[1]
    p3 = p3_ref[0] + p3_ref[1]
    col = lax.broadcasted_iota(i32, (1, GW), 1)
    cnt = jnp.sum(jnp.where(col == 32, p3, 0.0), axis=1, keepdims=True)
    cnt = jnp.maximum(cnt, 1.0)
    m1 = jax.nn.relu(_bn_mat(g1_ref[...], b1_ref[...], p1 / cnt))
    m2 = jax.nn.relu(_bn_mat(g2_ref[...], b2_ref[...], p2 / cnt))
    m3 = jax.nn.relu(_bn_mat(g3_ref[...], b3_ref[...], p3 / cnt))
    u = (jnp.dot(m1, wp1_ref[...], preferred_element_type=f32)
         + jnp.dot(m2, wp2_ref[...], preferred_element_type=f32)
         + jnp.dot(m3, wp3_ref[...], preferred_element_type=f32)
         + bfc_ref[...])
    u = _bn_mat(gf_ref[...], bf_ref[...], u)
    o_ref[...] = jnp.dot(u, wf_ref[...], preferred_element_type=f32) \
        + bff_ref[...]


def _head(p1, p2, p3, bn1, bn2, bn3, wp1, wp2, wp3, bfc, bnfc, wf, bff):
    df, do = 178, 128
    return pl.pallas_call(
        _head_body,
        grid=(1,),
        in_specs=[_full((2, NG, GW)), _full((2, NG, GW)), _full((2, NG, GW)),
                  _full((1, GW)), _full((1, GW)),
                  _full((1, GW)), _full((1, GW)),
                  _full((1, GW)), _full((1, GW)),
                  _full((GW, df)), _full((GW, df)), _full((GW, df)),
                  _full((1, df)), _full((1, df)), _full((1, df)),
                  _full((df, do)), _full((1, do))],
        out_specs=_full((NG, do)),
        out_shape=jax.ShapeDtypeStruct((NG, do), f32),
        name="tc_head",
    )(p1, p2, p3, bn1[0], bn1[1], bn2[0], bn2[1], bn3[0], bn3[1],
      wp1, wp2, wp3, bfc, bnfc[0], bnfc[1], wf, bff)


def _pad128(v, rows=None):
    out = jnp.zeros((rows or v.shape[0], GW), f32)
    return out.at[:v.shape[0], :v.shape[1]].set(v)


def _egnn_block(p, h_in, xp, row_g, col_g, row_s, zacc):
    h = _emb(h_in, p['emb_in']['W'], p['emb_in']['b'])
    for lp in p['layers']:
        w1 = lp['e1']['W']
        a, b = _prep(h, xp, w1[:H], w1[H:2 * H], lp['e1']['b'])
        ex = _gather_edges(a, b, row_g, col_g)
        w1c128 = _pad128(w1[2 * H:2 * H + 1], 1)
        w2p = jnp.concatenate([lp['e2']['W'],
                               jnp.zeros((GW - H, H), f32)], axis=0)
        ety = _edge(ex, w1c128, w2p, lp['e2']['b'].reshape(1, -1),
                    lp['att']['W'].reshape(1, -1),
                    lp['att']['b'].reshape(1, 1),
                    lp['c1']['W'], lp['c1']['b'].reshape(1, -1),
                    lp['c2']['W'].reshape(1, -1))
        part = _scatter_edges(ety, row_s, zacc)
        wn = lp['n1']['W']
        w1a128 = jnp.concatenate([wn[H:], jnp.zeros((GW - H, H), f32)], axis=0)
        h, xp = _node(h, xp, part, wn[:H], w1a128, lp['n1']['b'],
                      lp['n2']['W'], lp['n2']['b'])
    return h, xp


def kernel(x, coords, params, edge_index, batch):
    row, col = edge_index[0], edge_index[1]
    epad = EPAD - E
    row_g = jnp.concatenate([row, jnp.zeros((epad,), i32)]
                            ).reshape(NW, NCH, CH)
    col_g = jnp.concatenate([col, jnp.zeros((epad,), i32)]
                            ).reshape(NW, NCH, CH)
    row_s = jnp.concatenate([row, jnp.full((epad,), N + 8, i32)]
                            ).reshape(NW, NCH, CH)
    xp = jnp.pad(coords, ((0, 0), (0, 5)))
    zacc = jnp.zeros((NPS, GW), f32)
    zpool = jnp.zeros((NG, GW), f32)
    bpad = jnp.concatenate([batch, jnp.zeros((NPOOL - N,), i32)]
                           ).reshape(NW, PCH, CH)

    p = params
    h1, xp1 = _egnn_block(p['egnn1'], x, xp, row_g, col_g, row_s, zacc)
    h1 = _emb(h1, p['egnn1']['emb_out']['W'], p['egnn1']['emb_out']['b'])
    h2, xp2 = _egnn_block(p['egnn2'], h1, xp1, row_g, col_g, row_s, zacc)
    h2p = _emb(h2, p['egnn2']['emb_out']['W'], p['egnn2']['emb_out']['b'],
               mode='pad128')
    h3, _ = _egnn_block(p['egnn4'], h2p[:, :H], xp2, row_g, col_g, row_s,
                        zacc)
    h3p = _emb(h3, p['egnn4']['emb_out']['W'], p['egnn4']['emb_out']['b'],
               mode='ones128')

    v1 = jnp.zeros((NPOOL, GW), f32).at[:N].set(h1)
    v2 = jnp.zeros((NPOOL, GW), f32).at[:N].set(h2p)
    v3 = jnp.zeros((NPOOL, GW), f32).at[:N].set(h3p)
    p1 = _pool_scatter(v1, bpad, zpool)
    p2 = _pool_scatter(v2, bpad, zpool)
    p3 = _pool_scatter(v3, bpad, zpool)

    w = p['fc1']['W']
    wp1 = w[:128]
    wp2 = jnp.concatenate([w[128:192], jnp.zeros((GW - 64, 178), f32)])
    wp3 = jnp.concatenate([w[192:224], jnp.zeros((GW - 32, 178), f32)])
    out = _head(p1, p2, p3,
                (_pad128(p['bn1']['g'].reshape(1, -1), 1),
                 _pad128(p['bn1']['b'].reshape(1, -1), 1)),
                (_pad128(p['bn2']['g'].reshape(1, -1), 1),
                 _pad128(p['bn2']['b'].reshape(1, -1), 1)),
                (_pad128(p['bn3']['g'].reshape(1, -1), 1),
                 _pad128(p['bn3']['b'].reshape(1, -1), 1)),
                wp1, wp2, wp3, p['fc1']['b'].reshape(1, -1),
                (p['bn_fc1']['g'].reshape(1, -1),
                 p['bn_fc1']['b'].reshape(1, -1)),
                p['final']['W'], p['final']['b'].reshape(1, -1))
    return out
